# baseline (speedup 1.0000x reference)
import functools

import jax
import jax.numpy as jnp
from jax import lax
from jax.experimental import pallas as pl
from jax.experimental.pallas import tpu as pltpu
from jax.experimental.pallas import tpu_sc as plsc

N = 10000
E = 160000
IN_DIM = 128
OUT_DIM = 16
HID = 256

NC = 2
NS = 16
NW = NC * NS

E_PER_W = E // NW
E_PER_CORE = E // NC
E_PER_TILE = E_PER_CORE // NS
N_PAD = 10240
N_PER_TILE = N_PAD // NS

GCHUNK = 200
SCHUNK = 1000

EB = 2000
NB = 1000


def _elu(v):
    return jnp.where(v > 0, v, jnp.exp(v) - 1.0)


N_GCH = E_PER_W // GCHUNK


def _sc_gather_body(x_hbm, src_hbm, out_hbm, idx_v, rows_v, sem_g, sem_o):
    c = lax.axis_index("c")
    s = lax.axis_index("s")
    wid = s * NC + c
    base = wid * E_PER_W

    pltpu.sync_copy(src_hbm.at[pl.ds(base, E_PER_W)], idx_v)
    gd = [None] * N_GCH
    od = [None] * N_GCH
    for k in range(N_GCH):
        slot = k % 2
        if k >= 2:
            od[k - 2].wait()
        gd[k] = pltpu.async_copy(
            x_hbm.at[idx_v.at[pl.ds(k * GCHUNK, GCHUNK)]],
            rows_v.at[slot], sem_g)
        if k >= 1:
            gd[k - 1].wait()
            od[k - 1] = pltpu.async_copy(
                rows_v.at[(k - 1) % 2],
                out_hbm.at[pl.ds(base + (k - 1) * GCHUNK, GCHUNK)], sem_o)
    gd[N_GCH - 1].wait()
    od[N_GCH - 1] = pltpu.async_copy(
        rows_v.at[1 - N_GCH % 2],
        out_hbm.at[pl.ds(base + (N_GCH - 1) * GCHUNK, GCHUNK)], sem_o)
    od[N_GCH - 2].wait()
    od[N_GCH - 1].wait()


_sc_gather = functools.partial(
    pl.kernel,
    out_type=jax.ShapeDtypeStruct((E, IN_DIM), jnp.float32),
    mesh=plsc.VectorSubcoreMesh(core_axis_name="c", subcore_axis_name="s"),
    scratch_types=[
        pltpu.VMEM((E_PER_W,), jnp.int32),
        pltpu.VMEM((2, GCHUNK, IN_DIM), jnp.float32),
        pltpu.SemaphoreType.DMA,
        pltpu.SemaphoreType.DMA,
    ],
)(_sc_gather_body)


def _sc_scatter_body(msg_hbm, dst_hbm, out_hbm, acc_sh, idx_v, val_v, zrow_v,
                     sem):
    c = lax.axis_index("c")
    s = lax.axis_index("s")

    def zfill(i, carry):
        zrow_v[i, :] = jnp.zeros((OUT_DIM,), jnp.float32)
        return carry

    lax.fori_loop(0, N_PER_TILE, zfill, 0)
    pltpu.sync_copy(zrow_v, acc_sh.at[pl.ds(s * N_PER_TILE, N_PER_TILE)])
    plsc.subcore_barrier()

    base = c * E_PER_CORE + s * E_PER_TILE

    def step(k, carry):
        off = base + k * SCHUNK
        pltpu.sync_copy(dst_hbm.at[pl.ds(off, SCHUNK)], idx_v)
        pltpu.sync_copy(msg_hbm.at[pl.ds(off, SCHUNK)], val_v)
        pltpu.sync_copy(val_v, acc_sh.at[idx_v], add=True)
        return carry

    lax.fori_loop(0, E_PER_TILE // SCHUNK, step, 0)
    plsc.subcore_barrier()
    pltpu.sync_copy(acc_sh.at[pl.ds(s * N_PER_TILE, N_PER_TILE)],
                    out_hbm.at[c, pl.ds(s * N_PER_TILE, N_PER_TILE)])


_sc_scatter = functools.partial(
    pl.kernel,
    out_type=jax.ShapeDtypeStruct((NC, N_PAD, OUT_DIM), jnp.float32),
    mesh=plsc.VectorSubcoreMesh(core_axis_name="c", subcore_axis_name="s"),
    compiler_params=pltpu.CompilerParams(use_tc_tiling_on_sc=False),
    scratch_types=[
        pltpu.VMEM_SHARED((N_PAD, OUT_DIM), jnp.float32),
        pltpu.VMEM((SCHUNK,), jnp.int32),
        pltpu.VMEM((SCHUNK, OUT_DIM), jnp.float32),
        pltpu.VMEM((N_PER_TILE, OUT_DIM), jnp.float32),
        pltpu.SemaphoreType.DMA,
    ],
)(_sc_scatter_body)


def _sc_gs_body(xin_hbm, src_hbm, dst_hbm, out_hbm, acc_sh, sidx_v, didx_v,
                val_v, zrow_v, sem):
    c = lax.axis_index("c")
    s = lax.axis_index("s")

    def zfill(i, carry):
        zrow_v[i, :] = jnp.zeros((OUT_DIM,), jnp.float32)
        return carry

    lax.fori_loop(0, N_PER_TILE, zfill, 0)
    pltpu.sync_copy(zrow_v, acc_sh.at[pl.ds(s * N_PER_TILE, N_PER_TILE)])
    plsc.subcore_barrier()

    base = c * E_PER_CORE + s * E_PER_TILE

    def step(k, carry):
        off = base + k * SCHUNK
        pltpu.sync_copy(src_hbm.at[pl.ds(off, SCHUNK)], sidx_v)
        pltpu.async_copy(xin_hbm.at[sidx_v], val_v, sem).wait()
        pltpu.sync_copy(dst_hbm.at[pl.ds(off, SCHUNK)], didx_v)
        pltpu.sync_copy(val_v, acc_sh.at[didx_v], add=True)
        return carry

    lax.fori_loop(0, E_PER_TILE // SCHUNK, step, 0)
    plsc.subcore_barrier()
    pltpu.sync_copy(acc_sh.at[pl.ds(s * N_PER_TILE, N_PER_TILE)],
                    out_hbm.at[c, pl.ds(s * N_PER_TILE, N_PER_TILE)])


_sc_gs = functools.partial(
    pl.kernel,
    out_type=jax.ShapeDtypeStruct((NC, N_PAD, OUT_DIM), jnp.float32),
    mesh=plsc.VectorSubcoreMesh(core_axis_name="c", subcore_axis_name="s"),
    compiler_params=pltpu.CompilerParams(use_tc_tiling_on_sc=False),
    scratch_types=[
        pltpu.VMEM_SHARED((N_PAD, OUT_DIM), jnp.float32),
        pltpu.VMEM((SCHUNK,), jnp.int32),
        pltpu.VMEM((SCHUNK,), jnp.int32),
        pltpu.VMEM((SCHUNK, OUT_DIM), jnp.float32),
        pltpu.VMEM((N_PER_TILE, OUT_DIM), jnp.float32),
        pltpu.SemaphoreType.DMA,
    ],
)(_sc_gs_body)


SB = EB // 2


def _edge_body(attr_ref, xsrc_ref, w1_ref, b1_ref, w2_ref, b2_ref, w3p_ref,
               b3p_ref, msg_ref):
    for p in range(EB // SB):
        a = attr_ref[p * SB:(p + 1) * SB, :]
        h = _elu(jnp.dot(a, w1_ref[...], preferred_element_type=jnp.float32)
                 + b1_ref[...])
        h = _elu(jnp.dot(h, w2_ref[...], preferred_element_type=jnp.float32)
                 + b2_ref[...])
        h = _elu(jnp.dot(h, w3p_ref[...], preferred_element_type=jnp.float32)
                 + b3p_ref[...])
        xs = xsrc_ref[p * SB:(p + 1) * SB, :]
        cols = []
        for o in range(OUT_DIM):
            cols.append(jnp.sum(xs * h[:, o * IN_DIM:(o + 1) * IN_DIM],
                                axis=1, keepdims=True))
        msg_ref[p * SB:(p + 1) * SB, :] = jnp.concatenate(cols, axis=1)


def _edge_msg(edge_attr, xsrc, w1, b1r, w2, b2r, w3p, b3pr):
    grid = (E // EB,)
    return pl.pallas_call(
        _edge_body,
        grid=grid,
        in_specs=[
            pl.BlockSpec((EB, 4), lambda i: (i, 0)),
            pl.BlockSpec((EB, IN_DIM), lambda i: (i, 0)),
            pl.BlockSpec((4, HID), lambda i: (0, 0)),
            pl.BlockSpec((1, HID), lambda i: (0, 0)),
            pl.BlockSpec((HID, 1024), lambda i: (0, 0)),
            pl.BlockSpec((1, 1024), lambda i: (0, 0)),
            pl.BlockSpec((1024, IN_DIM * OUT_DIM), lambda i: (0, 0)),
            pl.BlockSpec((1, IN_DIM * OUT_DIM), lambda i: (0, 0)),
        ],
        out_specs=pl.BlockSpec((EB, OUT_DIM), lambda i: (i, 0)),
        out_shape=jax.ShapeDtypeStruct((E, OUT_DIM), jnp.float32),
    )(edge_attr, xsrc, w1, b1r, w2, b2r, w3p, b3pr)


def _root_body(x_ref, agg_ref, wroot_ref, broot_ref, xc_ref, e_ref):
    xc = jnp.dot(x_ref[...], wroot_ref[...],
                 preferred_element_type=jnp.float32)
    xc = xc + agg_ref[0] + agg_ref[1] + broot_ref[...]
    xc_ref[...] = xc
    e_ref[...] = _elu(xc)


def _root(x, agg, wroot, brootr):
    grid = (N // NB,)
    return pl.pallas_call(
        _root_body,
        grid=grid,
        in_specs=[
            pl.BlockSpec((NB, IN_DIM), lambda i: (i, 0)),
            pl.BlockSpec((NC, NB, OUT_DIM), lambda i: (0, i, 0)),
            pl.BlockSpec((IN_DIM, OUT_DIM), lambda i: (0, 0)),
            pl.BlockSpec((1, OUT_DIM), lambda i: (0, 0)),
        ],
        out_specs=[
            pl.BlockSpec((NB, OUT_DIM), lambda i: (i, 0)),
            pl.BlockSpec((NB, OUT_DIM), lambda i: (i, 0)),
        ],
        out_shape=[
            jax.ShapeDtypeStruct((N, OUT_DIM), jnp.float32),
            jax.ShapeDtypeStruct((N, OUT_DIM), jnp.float32),
        ],
    )(x, agg, wroot, brootr)


def _gin_body(xin_ref, nagg_ref, a1_ref, c1_ref, a2_ref, c2_ref, a3_ref,
              c3_ref, out_ref, e_ref):
    h = xin_ref[...] + nagg_ref[0] + nagg_ref[1]
    h = _elu(jnp.dot(h, a1_ref[...], preferred_element_type=jnp.float32)
             + c1_ref[...])
    h = _elu(jnp.dot(h, a2_ref[...], preferred_element_type=jnp.float32)
             + c2_ref[...])
    h = jnp.dot(h, a3_ref[...], preferred_element_type=jnp.float32) \
        + c3_ref[...]
    out_ref[...] = h
    e_ref[...] = _elu(h)


def _gin(xin, nagg, a1, c1r, a2, c2r, a3, c3r):
    grid = (N // NB,)
    return pl.pallas_call(
        _gin_body,
        grid=grid,
        in_specs=[
            pl.BlockSpec((NB, OUT_DIM), lambda i: (i, 0)),
            pl.BlockSpec((NC, NB, OUT_DIM), lambda i: (0, i, 0)),
            pl.BlockSpec((OUT_DIM, HID), lambda i: (0, 0)),
            pl.BlockSpec((1, HID), lambda i: (0, 0)),
            pl.BlockSpec((HID, HID), lambda i: (0, 0)),
            pl.BlockSpec((1, HID), lambda i: (0, 0)),
            pl.BlockSpec((HID, OUT_DIM), lambda i: (0, 0)),
            pl.BlockSpec((1, OUT_DIM), lambda i: (0, 0)),
        ],
        out_specs=[
            pl.BlockSpec((NB, OUT_DIM), lambda i: (i, 0)),
            pl.BlockSpec((NB, OUT_DIM), lambda i: (i, 0)),
        ],
        out_shape=[
            jax.ShapeDtypeStruct((N, OUT_DIM), jnp.float32),
            jax.ShapeDtypeStruct((N, OUT_DIM), jnp.float32),
        ],
    )(xin, nagg, a1, c1r, a2, c2r, a3, c3r)


def kernel(x, edge_index, edge_attr, w1, b1, w2, b2, w3, b3, wroot, broot,
           g1_w1, g1_b1, g1_w2, g1_b2, g1_w3, g1_b3, g2_w1, g2_b1, g2_w2,
           g2_b2, g2_w3, g2_b3):
    src = edge_index[0]
    dst = edge_index[1]

    w3p = w3.reshape(1024, IN_DIM, OUT_DIM).transpose(0, 2, 1) \
        .reshape(1024, IN_DIM * OUT_DIM)
    b3p = b3.reshape(IN_DIM, OUT_DIM).T.reshape(1, IN_DIM * OUT_DIM)

    xsrc = _sc_gather(x, src)
    msg = _edge_msg(edge_attr, xsrc, w1, b1.reshape(1, -1), w2,
                    b2.reshape(1, -1), w3p, b3p)
    agg = _sc_scatter(msg, dst)
    xc0, e0 = _root(x, agg, wroot, broot.reshape(1, -1))

    nagg1 = _sc_gs(e0, src, dst)
    xc1, e1 = _gin(e0, nagg1, g1_w1, g1_b1.reshape(1, -1), g1_w2,
                   g1_b2.reshape(1, -1), g1_w3, g1_b3.reshape(1, -1))

    nagg2 = _sc_gs(e1, src, dst)
    xc2, _ = _gin(e1, nagg2, g2_w1, g2_b1.reshape(1, -1), g2_w2,
                  g2_b2.reshape(1, -1), g2_w3, g2_b3.reshape(1, -1))

    return jnp.stack([xc0, xc1, xc2], axis=2)


# speedup vs baseline: 1.0062x; 1.0062x over previous
import functools

import jax
import jax.numpy as jnp
from jax import lax
from jax.experimental import pallas as pl
from jax.experimental.pallas import tpu as pltpu
from jax.experimental.pallas import tpu_sc as plsc

N = 10000
E = 160000
IN_DIM = 128
OUT_DIM = 16
HID = 256

NC = 2
NS = 16
NW = NC * NS

EA = 32000
EBB = E - EA

E_PER_CORE = E // NC
E_PER_TILE = E_PER_CORE // NS
N_PAD = 10240
N_PER_TILE = N_PAD // NS

GCHUNK = 200
SCHUNK = 1000

EB = 2000
NB = 1000

_MESH = plsc.VectorSubcoreMesh(core_axis_name="c", subcore_axis_name="s")


def _elu(v):
    return jnp.where(v > 0, v, jnp.exp(v) - 1.0)


def _make_gather(ne, base0):
    e_w = ne // NW
    n_ch = e_w // GCHUNK

    def body(x_hbm, src_hbm, out_hbm, idx_v, rows_v, sem_g, sem_o):
        c = lax.axis_index("c")
        s = lax.axis_index("s")
        wid = s * NC + c
        lbase = wid * e_w
        pltpu.sync_copy(src_hbm.at[pl.ds(base0 + lbase, e_w)], idx_v)
        gd = [None] * n_ch
        od = [None] * n_ch
        for k in range(n_ch):
            if k >= 2:
                od[k - 2].wait()
            gd[k] = pltpu.async_copy(
                x_hbm.at[idx_v.at[pl.ds(k * GCHUNK, GCHUNK)]],
                rows_v.at[k % 2], sem_g)
            if k >= 1:
                gd[k - 1].wait()
                od[k - 1] = pltpu.async_copy(
                    rows_v.at[(k - 1) % 2],
                    out_hbm.at[pl.ds(lbase + (k - 1) * GCHUNK, GCHUNK)],
                    sem_o)
        gd[n_ch - 1].wait()
        od[n_ch - 1] = pltpu.async_copy(
            rows_v.at[(n_ch - 1) % 2],
            out_hbm.at[pl.ds(lbase + (n_ch - 1) * GCHUNK, GCHUNK)], sem_o)
        if n_ch >= 2:
            od[n_ch - 2].wait()
        od[n_ch - 1].wait()

    return functools.partial(
        pl.kernel,
        out_type=jax.ShapeDtypeStruct((ne, IN_DIM), jnp.float32),
        mesh=_MESH,
        scratch_types=[
            pltpu.VMEM((e_w,), jnp.int32),
            pltpu.VMEM((2, GCHUNK, IN_DIM), jnp.float32),
            pltpu.SemaphoreType.DMA,
            pltpu.SemaphoreType.DMA,
        ],
    )(body)


_gather_a = _make_gather(EA, 0)
_gather_b = _make_gather(EBB, EA)


def _make_scatter(ne, base0):
    e_core = ne // NC
    e_tile = e_core // NS
    n_ch = e_tile // SCHUNK

    def body(msg_hbm, dst_hbm, out_hbm, acc_sh, idx_v, val_v, zrow_v, sem):
        c = lax.axis_index("c")
        s = lax.axis_index("s")

        def zfill(i, carry):
            zrow_v[i, :] = jnp.zeros((OUT_DIM,), jnp.float32)
            return carry

        lax.fori_loop(0, N_PER_TILE, zfill, 0)
        pltpu.sync_copy(zrow_v, acc_sh.at[pl.ds(s * N_PER_TILE, N_PER_TILE)])
        plsc.subcore_barrier()

        lbase = c * e_core + s * e_tile

        def step(k, carry):
            off = lbase + k * SCHUNK
            pltpu.sync_copy(dst_hbm.at[pl.ds(base0 + off, SCHUNK)], idx_v)
            pltpu.sync_copy(msg_hbm.at[pl.ds(off, SCHUNK)], val_v)
            pltpu.sync_copy(val_v, acc_sh.at[idx_v], add=True)
            return carry

        lax.fori_loop(0, n_ch, step, 0)
        plsc.subcore_barrier()
        pltpu.sync_copy(acc_sh.at[pl.ds(s * N_PER_TILE, N_PER_TILE)],
                        out_hbm.at[c, pl.ds(s * N_PER_TILE, N_PER_TILE)])

    return functools.partial(
        pl.kernel,
        out_type=jax.ShapeDtypeStruct((NC, N_PAD, OUT_DIM), jnp.float32),
        mesh=_MESH,
        compiler_params=pltpu.CompilerParams(use_tc_tiling_on_sc=False),
        scratch_types=[
            pltpu.VMEM_SHARED((N_PAD, OUT_DIM), jnp.float32),
            pltpu.VMEM((SCHUNK,), jnp.int32),
            pltpu.VMEM((SCHUNK, OUT_DIM), jnp.float32),
            pltpu.VMEM((N_PER_TILE, OUT_DIM), jnp.float32),
            pltpu.SemaphoreType.DMA,
        ],
    )(body)


_scatter_a = _make_scatter(EA, 0)
_scatter_b = _make_scatter(EBB, EA)


def _sc_gs_body(xin_hbm, src_hbm, dst_hbm, out_hbm, acc_sh, sidx_v, didx_v,
                val_v, zrow_v, sem):
    c = lax.axis_index("c")
    s = lax.axis_index("s")

    def zfill(i, carry):
        zrow_v[i, :] = jnp.zeros((OUT_DIM,), jnp.float32)
        return carry

    lax.fori_loop(0, N_PER_TILE, zfill, 0)
    pltpu.sync_copy(zrow_v, acc_sh.at[pl.ds(s * N_PER_TILE, N_PER_TILE)])
    plsc.subcore_barrier()

    base = c * E_PER_CORE + s * E_PER_TILE

    def step(k, carry):
        off = base + k * SCHUNK
        pltpu.sync_copy(src_hbm.at[pl.ds(off, SCHUNK)], sidx_v)
        pltpu.async_copy(xin_hbm.at[sidx_v], val_v, sem).wait()
        pltpu.sync_copy(dst_hbm.at[pl.ds(off, SCHUNK)], didx_v)
        pltpu.sync_copy(val_v, acc_sh.at[didx_v], add=True)
        return carry

    lax.fori_loop(0, E_PER_TILE // SCHUNK, step, 0)
    plsc.subcore_barrier()
    pltpu.sync_copy(acc_sh.at[pl.ds(s * N_PER_TILE, N_PER_TILE)],
                    out_hbm.at[c, pl.ds(s * N_PER_TILE, N_PER_TILE)])


_sc_gs = functools.partial(
    pl.kernel,
    out_type=jax.ShapeDtypeStruct((NC, N_PAD, OUT_DIM), jnp.float32),
    mesh=_MESH,
    compiler_params=pltpu.CompilerParams(use_tc_tiling_on_sc=False),
    scratch_types=[
        pltpu.VMEM_SHARED((N_PAD, OUT_DIM), jnp.float32),
        pltpu.VMEM((SCHUNK,), jnp.int32),
        pltpu.VMEM((SCHUNK,), jnp.int32),
        pltpu.VMEM((SCHUNK, OUT_DIM), jnp.float32),
        pltpu.VMEM((N_PER_TILE, OUT_DIM), jnp.float32),
        pltpu.SemaphoreType.DMA,
    ],
)(_sc_gs_body)


SB = EB // 2


def _edge_body(attr_ref, xsrc_ref, w1_ref, b1_ref, w2_ref, b2_ref, w3p_ref,
               b3p_ref, msg_ref):
    for p in range(EB // SB):
        a = attr_ref[p * SB:(p + 1) * SB, :]
        h = _elu(jnp.dot(a, w1_ref[...], preferred_element_type=jnp.float32)
                 + b1_ref[...])
        h = _elu(jnp.dot(h, w2_ref[...], preferred_element_type=jnp.float32)
                 + b2_ref[...])
        h = _elu(jnp.dot(h, w3p_ref[...], preferred_element_type=jnp.float32)
                 + b3p_ref[...])
        xs = xsrc_ref[p * SB:(p + 1) * SB, :]
        cols = []
        for o in range(OUT_DIM):
            cols.append(jnp.sum(xs * h[:, o * IN_DIM:(o + 1) * IN_DIM],
                                axis=1, keepdims=True))
        msg_ref[p * SB:(p + 1) * SB, :] = jnp.concatenate(cols, axis=1)


def _make_edge(ne, base0):
    blk0 = base0 // EB

    def call(edge_attr, xsrc, w1, b1r, w2, b2r, w3p, b3pr):
        return pl.pallas_call(
            _edge_body,
            grid=(ne // EB,),
            in_specs=[
                pl.BlockSpec((EB, 4), lambda i: (i + blk0, 0)),
                pl.BlockSpec((EB, IN_DIM), lambda i: (i, 0)),
                pl.BlockSpec((4, HID), lambda i: (0, 0)),
                pl.BlockSpec((1, HID), lambda i: (0, 0)),
                pl.BlockSpec((HID, 1024), lambda i: (0, 0)),
                pl.BlockSpec((1, 1024), lambda i: (0, 0)),
                pl.BlockSpec((1024, IN_DIM * OUT_DIM), lambda i: (0, 0)),
                pl.BlockSpec((1, IN_DIM * OUT_DIM), lambda i: (0, 0)),
            ],
            out_specs=pl.BlockSpec((EB, OUT_DIM), lambda i: (i, 0)),
            out_shape=jax.ShapeDtypeStruct((ne, OUT_DIM), jnp.float32),
        )(edge_attr, xsrc, w1, b1r, w2, b2r, w3p, b3pr)

    return call


_edge_a = _make_edge(EA, 0)
_edge_b = _make_edge(EBB, EA)


def _root_body(x_ref, agga_ref, aggb_ref, wroot_ref, broot_ref, xc_ref,
               e_ref):
    xc = jnp.dot(x_ref[...], wroot_ref[...],
                 preferred_element_type=jnp.float32)
    xc = xc + agga_ref[0] + agga_ref[1] + aggb_ref[0] + aggb_ref[1] \
        + broot_ref[...]
    xc_ref[...] = xc
    e_ref[...] = _elu(xc)


def _root(x, agga, aggb, wroot, brootr):
    grid = (N // NB,)
    return pl.pallas_call(
        _root_body,
        grid=grid,
        in_specs=[
            pl.BlockSpec((NB, IN_DIM), lambda i: (i, 0)),
            pl.BlockSpec((NC, NB, OUT_DIM), lambda i: (0, i, 0)),
            pl.BlockSpec((NC, NB, OUT_DIM), lambda i: (0, i, 0)),
            pl.BlockSpec((IN_DIM, OUT_DIM), lambda i: (0, 0)),
            pl.BlockSpec((1, OUT_DIM), lambda i: (0, 0)),
        ],
        out_specs=[
            pl.BlockSpec((NB, OUT_DIM), lambda i: (i, 0)),
            pl.BlockSpec((NB, OUT_DIM), lambda i: (i, 0)),
        ],
        out_shape=[
            jax.ShapeDtypeStruct((N, OUT_DIM), jnp.float32),
            jax.ShapeDtypeStruct((N, OUT_DIM), jnp.float32),
        ],
    )(x, agga, aggb, wroot, brootr)


def _gin_body(xin_ref, nagg_ref, a1_ref, c1_ref, a2_ref, c2_ref, a3_ref,
              c3_ref, out_ref, e_ref):
    h = xin_ref[...] + nagg_ref[0] + nagg_ref[1]
    h = _elu(jnp.dot(h, a1_ref[...], preferred_element_type=jnp.float32)
             + c1_ref[...])
    h = _elu(jnp.dot(h, a2_ref[...], preferred_element_type=jnp.float32)
             + c2_ref[...])
    h = jnp.dot(h, a3_ref[...], preferred_element_type=jnp.float32) \
        + c3_ref[...]
    out_ref[...] = h
    e_ref[...] = _elu(h)


def _gin(xin, nagg, a1, c1r, a2, c2r, a3, c3r):
    grid = (N // NB,)
    return pl.pallas_call(
        _gin_body,
        grid=grid,
        in_specs=[
            pl.BlockSpec((NB, OUT_DIM), lambda i: (i, 0)),
            pl.BlockSpec((NC, NB, OUT_DIM), lambda i: (0, i, 0)),
            pl.BlockSpec((OUT_DIM, HID), lambda i: (0, 0)),
            pl.BlockSpec((1, HID), lambda i: (0, 0)),
            pl.BlockSpec((HID, HID), lambda i: (0, 0)),
            pl.BlockSpec((1, HID), lambda i: (0, 0)),
            pl.BlockSpec((HID, OUT_DIM), lambda i: (0, 0)),
            pl.BlockSpec((1, OUT_DIM), lambda i: (0, 0)),
        ],
        out_specs=[
            pl.BlockSpec((NB, OUT_DIM), lambda i: (i, 0)),
            pl.BlockSpec((NB, OUT_DIM), lambda i: (i, 0)),
        ],
        out_shape=[
            jax.ShapeDtypeStruct((N, OUT_DIM), jnp.float32),
            jax.ShapeDtypeStruct((N, OUT_DIM), jnp.float32),
        ],
    )(xin, nagg, a1, c1r, a2, c2r, a3, c3r)


def kernel(x, edge_index, edge_attr, w1, b1, w2, b2, w3, b3, wroot, broot,
           g1_w1, g1_b1, g1_w2, g1_b2, g1_w3, g1_b3, g2_w1, g2_b1, g2_w2,
           g2_b2, g2_w3, g2_b3):
    src = edge_index[0]
    dst = edge_index[1]

    w3p = w3.reshape(1024, IN_DIM, OUT_DIM).transpose(0, 2, 1) \
        .reshape(1024, IN_DIM * OUT_DIM)
    b3p = b3.reshape(IN_DIM, OUT_DIM).T.reshape(1, IN_DIM * OUT_DIM)
    b1r = b1.reshape(1, -1)
    b2r = b2.reshape(1, -1)

    xsa = _gather_a(x, src)
    msga = _edge_a(edge_attr, xsa, w1, b1r, w2, b2r, w3p, b3p)
    agga = _scatter_a(msga, dst)
    xsb = _gather_b(x, src)
    msgb = _edge_b(edge_attr, xsb, w1, b1r, w2, b2r, w3p, b3p)
    aggb = _scatter_b(msgb, dst)
    xc0, e0 = _root(x, agga, aggb, wroot, broot.reshape(1, -1))

    nagg1 = _sc_gs(e0, src, dst)
    xc1, e1 = _gin(e0, nagg1, g1_w1, g1_b1.reshape(1, -1), g1_w2,
                   g1_b2.reshape(1, -1), g1_w3, g1_b3.reshape(1, -1))

    nagg2 = _sc_gs(e1, src, dst)
    xc2, _ = _gin(e1, nagg2, g2_w1, g2_b1.reshape(1, -1), g2_w2,
                  g2_b2.reshape(1, -1), g2_w3, g2_b3.reshape(1, -1))

    return jnp.stack([xc0, xc1, xc2], axis=2)


# speedup vs baseline: 1.0118x; 1.0055x over previous
import functools

import jax
import jax.numpy as jnp
from jax import lax
from jax.experimental import pallas as pl
from jax.experimental.pallas import tpu as pltpu
from jax.experimental.pallas import tpu_sc as plsc

N = 10000
E = 160000
IN_DIM = 128
OUT_DIM = 16
HID = 256

NC = 2
NS = 16
NW = NC * NS

EA = 32000
EBB = E - EA

E_PER_CORE = E // NC
E_PER_TILE = E_PER_CORE // NS
N_PAD = 10240
N_PER_TILE = N_PAD // NS

GCHUNK = 200
SCHUNK = 1000

EB = 2000
NB = 1000

_MESH = plsc.VectorSubcoreMesh(core_axis_name="c", subcore_axis_name="s")


def _elu(v):
    return jnp.where(v > 0, v, jnp.exp(v) - 1.0)


def _make_gather(ne, base0):
    e_w = ne // NW
    n_ch = e_w // GCHUNK

    def body(x_hbm, src_hbm, out_hbm, idx_v, rows_v, sem_g, sem_o):
        c = lax.axis_index("c")
        s = lax.axis_index("s")
        wid = s * NC + c
        lbase = wid * e_w
        pltpu.sync_copy(src_hbm.at[pl.ds(base0 + lbase, e_w)], idx_v)
        gd = [None] * n_ch
        od = [None] * n_ch
        for k in range(n_ch):
            if k >= 2:
                od[k - 2].wait()
            gd[k] = pltpu.async_copy(
                x_hbm.at[idx_v.at[pl.ds(k * GCHUNK, GCHUNK)]],
                rows_v.at[k % 2], sem_g)
            if k >= 1:
                gd[k - 1].wait()
                od[k - 1] = pltpu.async_copy(
                    rows_v.at[(k - 1) % 2],
                    out_hbm.at[pl.ds(lbase + (k - 1) * GCHUNK, GCHUNK)],
                    sem_o)
        gd[n_ch - 1].wait()
        od[n_ch - 1] = pltpu.async_copy(
            rows_v.at[(n_ch - 1) % 2],
            out_hbm.at[pl.ds(lbase + (n_ch - 1) * GCHUNK, GCHUNK)], sem_o)
        if n_ch >= 2:
            od[n_ch - 2].wait()
        od[n_ch - 1].wait()

    return functools.partial(
        pl.kernel,
        out_type=jax.ShapeDtypeStruct((ne, IN_DIM), jnp.float32),
        mesh=_MESH,
        scratch_types=[
            pltpu.VMEM((e_w,), jnp.int32),
            pltpu.VMEM((2, GCHUNK, IN_DIM), jnp.float32),
            pltpu.SemaphoreType.DMA,
            pltpu.SemaphoreType.DMA,
        ],
    )(body)


_gather_a = _make_gather(EA, 0)
_gather_b = _make_gather(EBB, EA)


def _make_scatter(ne, base0):
    e_core = ne // NC
    e_tile = e_core // NS
    n_ch = e_tile // SCHUNK

    def body(msg_hbm, dst_hbm, out_hbm, acc_sh, idx_v, val_v, zrow_v, sem):
        c = lax.axis_index("c")
        s = lax.axis_index("s")

        def zfill(i, carry):
            zrow_v[i, :] = jnp.zeros((OUT_DIM,), jnp.float32)
            return carry

        lax.fori_loop(0, N_PER_TILE, zfill, 0)
        pltpu.sync_copy(zrow_v, acc_sh.at[pl.ds(s * N_PER_TILE, N_PER_TILE)])
        plsc.subcore_barrier()

        lbase = c * e_core + s * e_tile

        def step(k, carry):
            off = lbase + k * SCHUNK
            pltpu.sync_copy(dst_hbm.at[pl.ds(base0 + off, SCHUNK)], idx_v)
            pltpu.sync_copy(msg_hbm.at[pl.ds(off, SCHUNK)], val_v)
            pltpu.sync_copy(val_v, acc_sh.at[idx_v], add=True)
            return carry

        lax.fori_loop(0, n_ch, step, 0)
        plsc.subcore_barrier()
        pltpu.sync_copy(acc_sh.at[pl.ds(s * N_PER_TILE, N_PER_TILE)],
                        out_hbm.at[c, pl.ds(s * N_PER_TILE, N_PER_TILE)])

    return functools.partial(
        pl.kernel,
        out_type=jax.ShapeDtypeStruct((NC, N_PAD, OUT_DIM), jnp.float32),
        mesh=_MESH,
        compiler_params=pltpu.CompilerParams(use_tc_tiling_on_sc=False),
        scratch_types=[
            pltpu.VMEM_SHARED((N_PAD, OUT_DIM), jnp.float32),
            pltpu.VMEM((SCHUNK,), jnp.int32),
            pltpu.VMEM((SCHUNK, OUT_DIM), jnp.float32),
            pltpu.VMEM((N_PER_TILE, OUT_DIM), jnp.float32),
            pltpu.SemaphoreType.DMA,
        ],
    )(body)


_scatter_a = _make_scatter(EA, 0)
_scatter_b = _make_scatter(EBB, EA)


N_GSCH = E_PER_TILE // SCHUNK


def _sc_gs_body(xin_hbm, src_hbm, dst_hbm, out_hbm, acc_sh, sidx_v, didx_v,
                val_v, zrow_v, sem_g, sem_s):
    c = lax.axis_index("c")
    s = lax.axis_index("s")

    def zfill(i, carry):
        zrow_v[i, :] = jnp.zeros((OUT_DIM,), jnp.float32)
        return carry

    lax.fori_loop(0, N_PER_TILE, zfill, 0)
    pltpu.sync_copy(zrow_v, acc_sh.at[pl.ds(s * N_PER_TILE, N_PER_TILE)])

    base = c * E_PER_CORE + s * E_PER_TILE
    pltpu.sync_copy(src_hbm.at[pl.ds(base, E_PER_TILE)], sidx_v)
    for k in range(N_GSCH):
        pltpu.sync_copy(dst_hbm.at[pl.ds(base + k * SCHUNK, SCHUNK)],
                        didx_v.at[k])
    plsc.subcore_barrier()

    gd = [None] * N_GSCH
    sd = [None] * N_GSCH
    for k in range(N_GSCH):
        if k >= 2:
            sd[k - 2].wait()
        gd[k] = pltpu.async_copy(
            xin_hbm.at[sidx_v.at[pl.ds(k * SCHUNK, SCHUNK)]],
            val_v.at[k % 2], sem_g)
        if k >= 1:
            gd[k - 1].wait()
            sd[k - 1] = pltpu.async_copy(
                val_v.at[(k - 1) % 2], acc_sh.at[didx_v.at[k - 1]], sem_s,
                add=True)
    gd[N_GSCH - 1].wait()
    sd[N_GSCH - 1] = pltpu.async_copy(
        val_v.at[(N_GSCH - 1) % 2], acc_sh.at[didx_v.at[N_GSCH - 1]], sem_s,
        add=True)
    if N_GSCH >= 2:
        sd[N_GSCH - 2].wait()
    sd[N_GSCH - 1].wait()

    plsc.subcore_barrier()
    pltpu.sync_copy(acc_sh.at[pl.ds(s * N_PER_TILE, N_PER_TILE)],
                    out_hbm.at[c, pl.ds(s * N_PER_TILE, N_PER_TILE)])


_sc_gs = functools.partial(
    pl.kernel,
    out_type=jax.ShapeDtypeStruct((NC, N_PAD, OUT_DIM), jnp.float32),
    mesh=_MESH,
    compiler_params=pltpu.CompilerParams(use_tc_tiling_on_sc=False),
    scratch_types=[
        pltpu.VMEM_SHARED((N_PAD, OUT_DIM), jnp.float32),
        pltpu.VMEM((E_PER_TILE,), jnp.int32),
        pltpu.VMEM((N_GSCH, SCHUNK), jnp.int32),
        pltpu.VMEM((2, SCHUNK, OUT_DIM), jnp.float32),
        pltpu.VMEM((N_PER_TILE, OUT_DIM), jnp.float32),
        pltpu.SemaphoreType.DMA,
        pltpu.SemaphoreType.DMA,
    ],
)(_sc_gs_body)


SB = EB // 2


def _edge_body(attr_ref, xsrc_ref, w1_ref, b1_ref, w2_ref, b2_ref, w3p_ref,
               b3p_ref, msg_ref):
    for p in range(EB // SB):
        a = attr_ref[p * SB:(p + 1) * SB, :]
        h = _elu(jnp.dot(a, w1_ref[...], preferred_element_type=jnp.float32)
                 + b1_ref[...])
        h = _elu(jnp.dot(h, w2_ref[...], preferred_element_type=jnp.float32)
                 + b2_ref[...])
        h = _elu(jnp.dot(h, w3p_ref[...], preferred_element_type=jnp.float32)
                 + b3p_ref[...])
        xs = xsrc_ref[p * SB:(p + 1) * SB, :]
        cols = []
        for o in range(OUT_DIM):
            cols.append(jnp.sum(xs * h[:, o * IN_DIM:(o + 1) * IN_DIM],
                                axis=1, keepdims=True))
        msg_ref[p * SB:(p + 1) * SB, :] = jnp.concatenate(cols, axis=1)


def _make_edge(ne, base0):
    blk0 = base0 // EB

    def call(edge_attr, xsrc, w1, b1r, w2, b2r, w3p, b3pr):
        return pl.pallas_call(
            _edge_body,
            grid=(ne // EB,),
            in_specs=[
                pl.BlockSpec((EB, 4), lambda i: (i + blk0, 0)),
                pl.BlockSpec((EB, IN_DIM), lambda i: (i, 0)),
                pl.BlockSpec((4, HID), lambda i: (0, 0)),
                pl.BlockSpec((1, HID), lambda i: (0, 0)),
                pl.BlockSpec((HID, 1024), lambda i: (0, 0)),
                pl.BlockSpec((1, 1024), lambda i: (0, 0)),
                pl.BlockSpec((1024, IN_DIM * OUT_DIM), lambda i: (0, 0)),
                pl.BlockSpec((1, IN_DIM * OUT_DIM), lambda i: (0, 0)),
            ],
            out_specs=pl.BlockSpec((EB, OUT_DIM), lambda i: (i, 0)),
            out_shape=jax.ShapeDtypeStruct((ne, OUT_DIM), jnp.float32),
        )(edge_attr, xsrc, w1, b1r, w2, b2r, w3p, b3pr)

    return call


_edge_a = _make_edge(EA, 0)
_edge_b = _make_edge(EBB, EA)


def _root_body(x_ref, agga_ref, aggb_ref, wroot_ref, broot_ref, xc_ref,
               e_ref):
    xc = jnp.dot(x_ref[...], wroot_ref[...],
                 preferred_element_type=jnp.float32)
    xc = xc + agga_ref[0] + agga_ref[1] + aggb_ref[0] + aggb_ref[1] \
        + broot_ref[...]
    xc_ref[...] = xc
    e_ref[...] = _elu(xc)


def _root(x, agga, aggb, wroot, brootr):
    grid = (N // NB,)
    return pl.pallas_call(
        _root_body,
        grid=grid,
        in_specs=[
            pl.BlockSpec((NB, IN_DIM), lambda i: (i, 0)),
            pl.BlockSpec((NC, NB, OUT_DIM), lambda i: (0, i, 0)),
            pl.BlockSpec((NC, NB, OUT_DIM), lambda i: (0, i, 0)),
            pl.BlockSpec((IN_DIM, OUT_DIM), lambda i: (0, 0)),
            pl.BlockSpec((1, OUT_DIM), lambda i: (0, 0)),
        ],
        out_specs=[
            pl.BlockSpec((NB, OUT_DIM), lambda i: (i, 0)),
            pl.BlockSpec((NB, OUT_DIM), lambda i: (i, 0)),
        ],
        out_shape=[
            jax.ShapeDtypeStruct((N, OUT_DIM), jnp.float32),
            jax.ShapeDtypeStruct((N, OUT_DIM), jnp.float32),
        ],
    )(x, agga, aggb, wroot, brootr)


def _gin_body(xin_ref, nagg_ref, a1_ref, c1_ref, a2_ref, c2_ref, a3_ref,
              c3_ref, out_ref, e_ref):
    h = xin_ref[...] + nagg_ref[0] + nagg_ref[1]
    h = _elu(jnp.dot(h, a1_ref[...], preferred_element_type=jnp.float32)
             + c1_ref[...])
    h = _elu(jnp.dot(h, a2_ref[...], preferred_element_type=jnp.float32)
             + c2_ref[...])
    h = jnp.dot(h, a3_ref[...], preferred_element_type=jnp.float32) \
        + c3_ref[...]
    out_ref[...] = h
    e_ref[...] = _elu(h)


def _gin(xin, nagg, a1, c1r, a2, c2r, a3, c3r):
    grid = (N // NB,)
    return pl.pallas_call(
        _gin_body,
        grid=grid,
        in_specs=[
            pl.BlockSpec((NB, OUT_DIM), lambda i: (i, 0)),
            pl.BlockSpec((NC, NB, OUT_DIM), lambda i: (0, i, 0)),
            pl.BlockSpec((OUT_DIM, HID), lambda i: (0, 0)),
            pl.BlockSpec((1, HID), lambda i: (0, 0)),
            pl.BlockSpec((HID, HID), lambda i: (0, 0)),
            pl.BlockSpec((1, HID), lambda i: (0, 0)),
            pl.BlockSpec((HID, OUT_DIM), lambda i: (0, 0)),
            pl.BlockSpec((1, OUT_DIM), lambda i: (0, 0)),
        ],
        out_specs=[
            pl.BlockSpec((NB, OUT_DIM), lambda i: (i, 0)),
            pl.BlockSpec((NB, OUT_DIM), lambda i: (i, 0)),
        ],
        out_shape=[
            jax.ShapeDtypeStruct((N, OUT_DIM), jnp.float32),
            jax.ShapeDtypeStruct((N, OUT_DIM), jnp.float32),
        ],
    )(xin, nagg, a1, c1r, a2, c2r, a3, c3r)


def kernel(x, edge_index, edge_attr, w1, b1, w2, b2, w3, b3, wroot, broot,
           g1_w1, g1_b1, g1_w2, g1_b2, g1_w3, g1_b3, g2_w1, g2_b1, g2_w2,
           g2_b2, g2_w3, g2_b3):
    src = edge_index[0]
    dst = edge_index[1]

    w3p = w3.reshape(1024, IN_DIM, OUT_DIM).transpose(0, 2, 1) \
        .reshape(1024, IN_DIM * OUT_DIM)
    b3p = b3.reshape(IN_DIM, OUT_DIM).T.reshape(1, IN_DIM * OUT_DIM)
    b1r = b1.reshape(1, -1)
    b2r = b2.reshape(1, -1)

    xsa = _gather_a(x, src)
    msga = _edge_a(edge_attr, xsa, w1, b1r, w2, b2r, w3p, b3p)
    agga = _scatter_a(msga, dst)
    xsb = _gather_b(x, src)
    msgb = _edge_b(edge_attr, xsb, w1, b1r, w2, b2r, w3p, b3p)
    aggb = _scatter_b(msgb, dst)
    xc0, e0 = _root(x, agga, aggb, wroot, broot.reshape(1, -1))

    nagg1 = _sc_gs(e0, src, dst)
    xc1, e1 = _gin(e0, nagg1, g1_w1, g1_b1.reshape(1, -1), g1_w2,
                   g1_b2.reshape(1, -1), g1_w3, g1_b3.reshape(1, -1))

    nagg2 = _sc_gs(e1, src, dst)
    xc2, _ = _gin(e1, nagg2, g2_w1, g2_b1.reshape(1, -1), g2_w2,
                  g2_b2.reshape(1, -1), g2_w3, g2_b3.reshape(1, -1))

    return jnp.stack([xc0, xc1, xc2], axis=2)


# speedup vs baseline: 1.0150x; 1.0032x over previous
import functools

import jax
import jax.numpy as jnp
from jax import lax
from jax.experimental import pallas as pl
from jax.experimental.pallas import tpu as pltpu
from jax.experimental.pallas import tpu_sc as plsc

N = 10000
E = 160000
IN_DIM = 128
OUT_DIM = 16
HID = 256

NC = 2
NS = 16
NW = NC * NS

EA = 32000
EBB = E - EA

E_PER_CORE = E // NC
E_PER_TILE = E_PER_CORE // NS
N_PAD = 10240
N_PER_TILE = N_PAD // NS

GCHUNK = 200
SCHUNK = 1000

EB = 2000
NB = 1000

_MESH = plsc.VectorSubcoreMesh(core_axis_name="c", subcore_axis_name="s")


def _elu(v):
    return jnp.where(v > 0, v, jnp.exp(v) - 1.0)


def _make_gather(ne, base0):
    e_w = ne // NW
    n_ch = e_w // GCHUNK

    def body(x_hbm, src_hbm, out_hbm, idx_v, rows_v, sem_g, sem_o):
        c = lax.axis_index("c")
        s = lax.axis_index("s")
        wid = s * NC + c
        lbase = wid * e_w
        pltpu.sync_copy(src_hbm.at[pl.ds(base0 + lbase, e_w)], idx_v)
        gd = [None] * n_ch
        od = [None] * n_ch
        for k in range(n_ch):
            if k >= 2:
                od[k - 2].wait()
            gd[k] = pltpu.async_copy(
                x_hbm.at[idx_v.at[pl.ds(k * GCHUNK, GCHUNK)]],
                rows_v.at[k % 2], sem_g)
            if k >= 1:
                gd[k - 1].wait()
                od[k - 1] = pltpu.async_copy(
                    rows_v.at[(k - 1) % 2],
                    out_hbm.at[pl.ds(lbase + (k - 1) * GCHUNK, GCHUNK)],
                    sem_o)
        gd[n_ch - 1].wait()
        od[n_ch - 1] = pltpu.async_copy(
            rows_v.at[(n_ch - 1) % 2],
            out_hbm.at[pl.ds(lbase + (n_ch - 1) * GCHUNK, GCHUNK)], sem_o)
        if n_ch >= 2:
            od[n_ch - 2].wait()
        od[n_ch - 1].wait()

    return functools.partial(
        pl.kernel,
        out_type=jax.ShapeDtypeStruct((ne, IN_DIM), jnp.float32),
        mesh=_MESH,
        scratch_types=[
            pltpu.VMEM((e_w,), jnp.int32),
            pltpu.VMEM((2, GCHUNK, IN_DIM), jnp.float32),
            pltpu.SemaphoreType.DMA,
            pltpu.SemaphoreType.DMA,
        ],
    )(body)


_gather_a = _make_gather(EA, 0)
_gather_b = _make_gather(EBB, EA)


def _make_scatter(ne, base0):
    e_core = ne // NC
    e_tile = e_core // NS
    n_ch = e_tile // SCHUNK

    def body(msg_hbm, dst_hbm, out_hbm, acc_sh, idx_v, val_v, zrow_v, sem_g,
             sem_s):
        c = lax.axis_index("c")
        s = lax.axis_index("s")

        def zfill(i, carry):
            zrow_v[i, :] = jnp.zeros((OUT_DIM,), jnp.float32)
            return carry

        lax.fori_loop(0, N_PER_TILE, zfill, 0)
        pltpu.sync_copy(zrow_v, acc_sh.at[pl.ds(s * N_PER_TILE, N_PER_TILE)])

        lbase = c * e_core + s * e_tile
        for k in range(n_ch):
            pltpu.sync_copy(
                dst_hbm.at[pl.ds(base0 + lbase + k * SCHUNK, SCHUNK)],
                idx_v.at[k])
        plsc.subcore_barrier()

        gd = [None] * n_ch
        sd = [None] * n_ch
        for k in range(n_ch):
            if k >= 2:
                sd[k - 2].wait()
            gd[k] = pltpu.async_copy(
                msg_hbm.at[pl.ds(lbase + k * SCHUNK, SCHUNK)],
                val_v.at[k % 2], sem_g)
            if k >= 1:
                gd[k - 1].wait()
                sd[k - 1] = pltpu.async_copy(
                    val_v.at[(k - 1) % 2], acc_sh.at[idx_v.at[k - 1]],
                    sem_s, add=True)
        gd[n_ch - 1].wait()
        sd[n_ch - 1] = pltpu.async_copy(
            val_v.at[(n_ch - 1) % 2], acc_sh.at[idx_v.at[n_ch - 1]], sem_s,
            add=True)
        if n_ch >= 2:
            sd[n_ch - 2].wait()
        sd[n_ch - 1].wait()

        plsc.subcore_barrier()
        pltpu.sync_copy(acc_sh.at[pl.ds(s * N_PER_TILE, N_PER_TILE)],
                        out_hbm.at[c, pl.ds(s * N_PER_TILE, N_PER_TILE)])

    return functools.partial(
        pl.kernel,
        out_type=jax.ShapeDtypeStruct((NC, N_PAD, OUT_DIM), jnp.float32),
        mesh=_MESH,
        compiler_params=pltpu.CompilerParams(use_tc_tiling_on_sc=False),
        scratch_types=[
            pltpu.VMEM_SHARED((N_PAD, OUT_DIM), jnp.float32),
            pltpu.VMEM((n_ch, SCHUNK), jnp.int32),
            pltpu.VMEM((2, SCHUNK, OUT_DIM), jnp.float32),
            pltpu.VMEM((N_PER_TILE, OUT_DIM), jnp.float32),
            pltpu.SemaphoreType.DMA,
            pltpu.SemaphoreType.DMA,
        ],
    )(body)


_scatter_a = _make_scatter(EA, 0)
_scatter_b = _make_scatter(EBB, EA)


N_GSCH = E_PER_TILE // SCHUNK


def _sc_gs_body(xin_hbm, src_hbm, dst_hbm, out_hbm, acc_sh, sidx_v, didx_v,
                val_v, zrow_v, sem_g, sem_s):
    c = lax.axis_index("c")
    s = lax.axis_index("s")

    def zfill(i, carry):
        zrow_v[i, :] = jnp.zeros((OUT_DIM,), jnp.float32)
        return carry

    lax.fori_loop(0, N_PER_TILE, zfill, 0)
    pltpu.sync_copy(zrow_v, acc_sh.at[pl.ds(s * N_PER_TILE, N_PER_TILE)])

    base = c * E_PER_CORE + s * E_PER_TILE
    pltpu.sync_copy(src_hbm.at[pl.ds(base, E_PER_TILE)], sidx_v)
    for k in range(N_GSCH):
        pltpu.sync_copy(dst_hbm.at[pl.ds(base + k * SCHUNK, SCHUNK)],
                        didx_v.at[k])
    plsc.subcore_barrier()

    gd = [None] * N_GSCH
    sd = [None] * N_GSCH
    for k in range(N_GSCH):
        if k >= 2:
            sd[k - 2].wait()
        gd[k] = pltpu.async_copy(
            xin_hbm.at[sidx_v.at[pl.ds(k * SCHUNK, SCHUNK)]],
            val_v.at[k % 2], sem_g)
        if k >= 1:
            gd[k - 1].wait()
            sd[k - 1] = pltpu.async_copy(
                val_v.at[(k - 1) % 2], acc_sh.at[didx_v.at[k - 1]], sem_s,
                add=True)
    gd[N_GSCH - 1].wait()
    sd[N_GSCH - 1] = pltpu.async_copy(
        val_v.at[(N_GSCH - 1) % 2], acc_sh.at[didx_v.at[N_GSCH - 1]], sem_s,
        add=True)
    if N_GSCH >= 2:
        sd[N_GSCH - 2].wait()
    sd[N_GSCH - 1].wait()

    plsc.subcore_barrier()
    pltpu.sync_copy(acc_sh.at[pl.ds(s * N_PER_TILE, N_PER_TILE)],
                    out_hbm.at[c, pl.ds(s * N_PER_TILE, N_PER_TILE)])


_sc_gs = functools.partial(
    pl.kernel,
    out_type=jax.ShapeDtypeStruct((NC, N_PAD, OUT_DIM), jnp.float32),
    mesh=_MESH,
    compiler_params=pltpu.CompilerParams(use_tc_tiling_on_sc=False),
    scratch_types=[
        pltpu.VMEM_SHARED((N_PAD, OUT_DIM), jnp.float32),
        pltpu.VMEM((E_PER_TILE,), jnp.int32),
        pltpu.VMEM((N_GSCH, SCHUNK), jnp.int32),
        pltpu.VMEM((2, SCHUNK, OUT_DIM), jnp.float32),
        pltpu.VMEM((N_PER_TILE, OUT_DIM), jnp.float32),
        pltpu.SemaphoreType.DMA,
        pltpu.SemaphoreType.DMA,
    ],
)(_sc_gs_body)


SB = EB // 2


def _edge_body(attr_ref, xsrc_ref, w1_ref, b1_ref, w2_ref, b2_ref, w3p_ref,
               b3p_ref, msg_ref):
    for p in range(EB // SB):
        a = attr_ref[p * SB:(p + 1) * SB, :]
        h = _elu(jnp.dot(a, w1_ref[...], preferred_element_type=jnp.float32)
                 + b1_ref[...])
        h = _elu(jnp.dot(h, w2_ref[...], preferred_element_type=jnp.float32)
                 + b2_ref[...])
        h = _elu(jnp.dot(h, w3p_ref[...], preferred_element_type=jnp.float32)
                 + b3p_ref[...])
        xs = xsrc_ref[p * SB:(p + 1) * SB, :]
        cols = []
        for o in range(OUT_DIM):
            cols.append(jnp.sum(xs * h[:, o * IN_DIM:(o + 1) * IN_DIM],
                                axis=1, keepdims=True))
        msg_ref[p * SB:(p + 1) * SB, :] = jnp.concatenate(cols, axis=1)


def _make_edge(ne, base0):
    blk0 = base0 // EB

    def call(edge_attr, xsrc, w1, b1r, w2, b2r, w3p, b3pr):
        return pl.pallas_call(
            _edge_body,
            grid=(ne // EB,),
            in_specs=[
                pl.BlockSpec((EB, 4), lambda i: (i + blk0, 0)),
                pl.BlockSpec((EB, IN_DIM), lambda i: (i, 0)),
                pl.BlockSpec((4, HID), lambda i: (0, 0)),
                pl.BlockSpec((1, HID), lambda i: (0, 0)),
                pl.BlockSpec((HID, 1024), lambda i: (0, 0)),
                pl.BlockSpec((1, 1024), lambda i: (0, 0)),
                pl.BlockSpec((1024, IN_DIM * OUT_DIM), lambda i: (0, 0)),
                pl.BlockSpec((1, IN_DIM * OUT_DIM), lambda i: (0, 0)),
            ],
            out_specs=pl.BlockSpec((EB, OUT_DIM), lambda i: (i, 0)),
            out_shape=jax.ShapeDtypeStruct((ne, OUT_DIM), jnp.float32),
        )(edge_attr, xsrc, w1, b1r, w2, b2r, w3p, b3pr)

    return call


_edge_a = _make_edge(EA, 0)
_edge_b = _make_edge(EBB, EA)


def _root_body(x_ref, agga_ref, aggb_ref, wroot_ref, broot_ref, xc_ref,
               e_ref):
    xc = jnp.dot(x_ref[...], wroot_ref[...],
                 preferred_element_type=jnp.float32)
    xc = xc + agga_ref[0] + agga_ref[1] + aggb_ref[0] + aggb_ref[1] \
        + broot_ref[...]
    xc_ref[...] = xc
    e_ref[...] = _elu(xc)


def _root(x, agga, aggb, wroot, brootr):
    grid = (N // NB,)
    return pl.pallas_call(
        _root_body,
        grid=grid,
        in_specs=[
            pl.BlockSpec((NB, IN_DIM), lambda i: (i, 0)),
            pl.BlockSpec((NC, NB, OUT_DIM), lambda i: (0, i, 0)),
            pl.BlockSpec((NC, NB, OUT_DIM), lambda i: (0, i, 0)),
            pl.BlockSpec((IN_DIM, OUT_DIM), lambda i: (0, 0)),
            pl.BlockSpec((1, OUT_DIM), lambda i: (0, 0)),
        ],
        out_specs=[
            pl.BlockSpec((NB, OUT_DIM), lambda i: (i, 0)),
            pl.BlockSpec((NB, OUT_DIM), lambda i: (i, 0)),
        ],
        out_shape=[
            jax.ShapeDtypeStruct((N, OUT_DIM), jnp.float32),
            jax.ShapeDtypeStruct((N, OUT_DIM), jnp.float32),
        ],
    )(x, agga, aggb, wroot, brootr)


def _gin_body(xin_ref, nagg_ref, a1_ref, c1_ref, a2_ref, c2_ref, a3_ref,
              c3_ref, out_ref, e_ref):
    h = xin_ref[...] + nagg_ref[0] + nagg_ref[1]
    h = _elu(jnp.dot(h, a1_ref[...], preferred_element_type=jnp.float32)
             + c1_ref[...])
    h = _elu(jnp.dot(h, a2_ref[...], preferred_element_type=jnp.float32)
             + c2_ref[...])
    h = jnp.dot(h, a3_ref[...], preferred_element_type=jnp.float32) \
        + c3_ref[...]
    out_ref[...] = h
    e_ref[...] = _elu(h)


def _gin(xin, nagg, a1, c1r, a2, c2r, a3, c3r):
    grid = (N // NB,)
    return pl.pallas_call(
        _gin_body,
        grid=grid,
        in_specs=[
            pl.BlockSpec((NB, OUT_DIM), lambda i: (i, 0)),
            pl.BlockSpec((NC, NB, OUT_DIM), lambda i: (0, i, 0)),
            pl.BlockSpec((OUT_DIM, HID), lambda i: (0, 0)),
            pl.BlockSpec((1, HID), lambda i: (0, 0)),
            pl.BlockSpec((HID, HID), lambda i: (0, 0)),
            pl.BlockSpec((1, HID), lambda i: (0, 0)),
            pl.BlockSpec((HID, OUT_DIM), lambda i: (0, 0)),
            pl.BlockSpec((1, OUT_DIM), lambda i: (0, 0)),
        ],
        out_specs=[
            pl.BlockSpec((NB, OUT_DIM), lambda i: (i, 0)),
            pl.BlockSpec((NB, OUT_DIM), lambda i: (i, 0)),
        ],
        out_shape=[
            jax.ShapeDtypeStruct((N, OUT_DIM), jnp.float32),
            jax.ShapeDtypeStruct((N, OUT_DIM), jnp.float32),
        ],
    )(xin, nagg, a1, c1r, a2, c2r, a3, c3r)


def kernel(x, edge_index, edge_attr, w1, b1, w2, b2, w3, b3, wroot, broot,
           g1_w1, g1_b1, g1_w2, g1_b2, g1_w3, g1_b3, g2_w1, g2_b1, g2_w2,
           g2_b2, g2_w3, g2_b3):
    src = edge_index[0]
    dst = edge_index[1]

    w3p = w3.reshape(1024, IN_DIM, OUT_DIM).transpose(0, 2, 1) \
        .reshape(1024, IN_DIM * OUT_DIM)
    b3p = b3.reshape(IN_DIM, OUT_DIM).T.reshape(1, IN_DIM * OUT_DIM)
    b1r = b1.reshape(1, -1)
    b2r = b2.reshape(1, -1)

    xsa = _gather_a(x, src)
    msga = _edge_a(edge_attr, xsa, w1, b1r, w2, b2r, w3p, b3p)
    agga = _scatter_a(msga, dst)
    xsb = _gather_b(x, src)
    msgb = _edge_b(edge_attr, xsb, w1, b1r, w2, b2r, w3p, b3p)
    aggb = _scatter_b(msgb, dst)
    xc0, e0 = _root(x, agga, aggb, wroot, broot.reshape(1, -1))

    nagg1 = _sc_gs(e0, src, dst)
    xc1, e1 = _gin(e0, nagg1, g1_w1, g1_b1.reshape(1, -1), g1_w2,
                   g1_b2.reshape(1, -1), g1_w3, g1_b3.reshape(1, -1))

    nagg2 = _sc_gs(e1, src, dst)
    xc2, _ = _gin(e1, nagg2, g2_w1, g2_b1.reshape(1, -1), g2_w2,
                  g2_b2.reshape(1, -1), g2_w3, g2_b3.reshape(1, -1))

    return jnp.stack([xc0, xc1, xc2], axis=2)


# speedup vs baseline: 1.0413x; 1.0259x over previous
import functools

import jax
import jax.numpy as jnp
from jax import lax
from jax.experimental import pallas as pl
from jax.experimental.pallas import tpu as pltpu
from jax.experimental.pallas import tpu_sc as plsc

N = 10000
E = 160000
IN_DIM = 128
OUT_DIM = 16
HID = 256

NC = 2
NS = 16
NW = NC * NS

EA = 32000
EBB = E - EA

E_PER_CORE = E // NC
E_PER_TILE = E_PER_CORE // NS
N_PAD = 10240
N_PER_TILE = N_PAD // NS

GCHUNK = 200
SCHUNK = 1000

EB = 2000
NB = 1000

_MESH = plsc.VectorSubcoreMesh(core_axis_name="c", subcore_axis_name="s")


def _elu(v):
    return jnp.where(v > 0, v, jnp.exp(v) - 1.0)


def _make_gather(ne, base0):
    e_w = ne // NW
    n_ch = e_w // GCHUNK

    def body(x_hbm, src_hbm, out_hbm, idx_v, rows_v, sem_g, sem_o):
        c = lax.axis_index("c")
        s = lax.axis_index("s")
        wid = s * NC + c
        lbase = wid * e_w
        pltpu.sync_copy(src_hbm.at[pl.ds(base0 + lbase, e_w)], idx_v)
        gd = [None] * n_ch
        od = [None] * n_ch
        for k in range(n_ch):
            if k >= 2:
                od[k - 2].wait()
            gd[k] = pltpu.async_copy(
                x_hbm.at[idx_v.at[pl.ds(k * GCHUNK, GCHUNK)]],
                rows_v.at[k % 2], sem_g)
            if k >= 1:
                gd[k - 1].wait()
                od[k - 1] = pltpu.async_copy(
                    rows_v.at[(k - 1) % 2],
                    out_hbm.at[pl.ds(lbase + (k - 1) * GCHUNK, GCHUNK)],
                    sem_o)
        gd[n_ch - 1].wait()
        od[n_ch - 1] = pltpu.async_copy(
            rows_v.at[(n_ch - 1) % 2],
            out_hbm.at[pl.ds(lbase + (n_ch - 1) * GCHUNK, GCHUNK)], sem_o)
        if n_ch >= 2:
            od[n_ch - 2].wait()
        od[n_ch - 1].wait()

    return functools.partial(
        pl.kernel,
        out_type=jax.ShapeDtypeStruct((ne, IN_DIM), jnp.float32),
        mesh=_MESH,
        scratch_types=[
            pltpu.VMEM((e_w,), jnp.int32),
            pltpu.VMEM((2, GCHUNK, IN_DIM), jnp.float32),
            pltpu.SemaphoreType.DMA,
            pltpu.SemaphoreType.DMA,
        ],
    )(body)


_gather_a = _make_gather(EA, 0)
_gather_b = _make_gather(EBB, EA)


def _make_scatter(ne, base0):
    e_core = ne // NC
    e_tile = e_core // NS
    n_ch = e_tile // SCHUNK

    def body(msg_hbm, dst_hbm, out_hbm, acc_sh, idx_v, val_v, zrow_v, sem_g,
             sem_s):
        c = lax.axis_index("c")
        s = lax.axis_index("s")

        def zfill(i, carry):
            zrow_v[i, :] = jnp.zeros((OUT_DIM,), jnp.float32)
            return carry

        lax.fori_loop(0, N_PER_TILE, zfill, 0)
        pltpu.sync_copy(zrow_v, acc_sh.at[pl.ds(s * N_PER_TILE, N_PER_TILE)])

        lbase = c * e_core + s * e_tile
        for k in range(n_ch):
            pltpu.sync_copy(
                dst_hbm.at[pl.ds(base0 + lbase + k * SCHUNK, SCHUNK)],
                idx_v.at[k])
        plsc.subcore_barrier()

        gd = [None] * n_ch
        sd = [None] * n_ch
        for k in range(n_ch):
            if k >= 2:
                sd[k - 2].wait()
            gd[k] = pltpu.async_copy(
                msg_hbm.at[pl.ds(lbase + k * SCHUNK, SCHUNK)],
                val_v.at[k % 2], sem_g)
            if k >= 1:
                gd[k - 1].wait()
                sd[k - 1] = pltpu.async_copy(
                    val_v.at[(k - 1) % 2], acc_sh.at[idx_v.at[k - 1]],
                    sem_s, add=True)
        gd[n_ch - 1].wait()
        sd[n_ch - 1] = pltpu.async_copy(
            val_v.at[(n_ch - 1) % 2], acc_sh.at[idx_v.at[n_ch - 1]], sem_s,
            add=True)
        if n_ch >= 2:
            sd[n_ch - 2].wait()
        sd[n_ch - 1].wait()

        plsc.subcore_barrier()
        pltpu.sync_copy(acc_sh.at[pl.ds(s * N_PER_TILE, N_PER_TILE)],
                        out_hbm.at[c, pl.ds(s * N_PER_TILE, N_PER_TILE)])

    return functools.partial(
        pl.kernel,
        out_type=jax.ShapeDtypeStruct((NC, N_PAD, OUT_DIM), jnp.float32),
        mesh=_MESH,
        compiler_params=pltpu.CompilerParams(use_tc_tiling_on_sc=False),
        scratch_types=[
            pltpu.VMEM_SHARED((N_PAD, OUT_DIM), jnp.float32),
            pltpu.VMEM((n_ch, SCHUNK), jnp.int32),
            pltpu.VMEM((2, SCHUNK, OUT_DIM), jnp.float32),
            pltpu.VMEM((N_PER_TILE, OUT_DIM), jnp.float32),
            pltpu.SemaphoreType.DMA,
            pltpu.SemaphoreType.DMA,
        ],
    )(body)


_scatter_a = _make_scatter(EA, 0)
_scatter_b = _make_scatter(EBB, EA)


N_GSCH = E_PER_TILE // SCHUNK


def _sc_gs_body(xin_hbm, src_hbm, dst_hbm, out_hbm, acc_sh, sidx_v, didx_v,
                val_v, zrow_v, sem_g, sem_s):
    c = lax.axis_index("c")
    s = lax.axis_index("s")

    def zfill(i, carry):
        zrow_v[i, :] = jnp.zeros((OUT_DIM,), jnp.float32)
        return carry

    lax.fori_loop(0, N_PER_TILE, zfill, 0)
    pltpu.sync_copy(zrow_v, acc_sh.at[pl.ds(s * N_PER_TILE, N_PER_TILE)])

    base = c * E_PER_CORE + s * E_PER_TILE
    pltpu.sync_copy(src_hbm.at[pl.ds(base, E_PER_TILE)], sidx_v)
    for k in range(N_GSCH):
        pltpu.sync_copy(dst_hbm.at[pl.ds(base + k * SCHUNK, SCHUNK)],
                        didx_v.at[k])
    plsc.subcore_barrier()

    gd = [None] * N_GSCH
    sd = [None] * N_GSCH
    for k in range(N_GSCH):
        if k >= 2:
            sd[k - 2].wait()
        gd[k] = pltpu.async_copy(
            xin_hbm.at[sidx_v.at[pl.ds(k * SCHUNK, SCHUNK)]],
            val_v.at[k % 2], sem_g)
        if k >= 1:
            gd[k - 1].wait()
            sd[k - 1] = pltpu.async_copy(
                val_v.at[(k - 1) % 2], acc_sh.at[didx_v.at[k - 1]], sem_s,
                add=True)
    gd[N_GSCH - 1].wait()
    sd[N_GSCH - 1] = pltpu.async_copy(
        val_v.at[(N_GSCH - 1) % 2], acc_sh.at[didx_v.at[N_GSCH - 1]], sem_s,
        add=True)
    if N_GSCH >= 2:
        sd[N_GSCH - 2].wait()
    sd[N_GSCH - 1].wait()

    plsc.subcore_barrier()
    pltpu.sync_copy(acc_sh.at[pl.ds(s * N_PER_TILE, N_PER_TILE)],
                    out_hbm.at[c, pl.ds(s * N_PER_TILE, N_PER_TILE)])


_sc_gs = functools.partial(
    pl.kernel,
    out_type=jax.ShapeDtypeStruct((NC, N_PAD, OUT_DIM), jnp.float32),
    mesh=_MESH,
    compiler_params=pltpu.CompilerParams(use_tc_tiling_on_sc=False),
    scratch_types=[
        pltpu.VMEM_SHARED((N_PAD, OUT_DIM), jnp.float32),
        pltpu.VMEM((E_PER_TILE,), jnp.int32),
        pltpu.VMEM((N_GSCH, SCHUNK), jnp.int32),
        pltpu.VMEM((2, SCHUNK, OUT_DIM), jnp.float32),
        pltpu.VMEM((N_PER_TILE, OUT_DIM), jnp.float32),
        pltpu.SemaphoreType.DMA,
        pltpu.SemaphoreType.DMA,
    ],
)(_sc_gs_body)


SB = EB // 2


def _edge_body(attr_ref, xsrc_ref, w1_ref, b1_ref, w2_ref, b2_ref, w3p_ref,
               b3p_ref, msg_ref):
    for p in range(EB // SB):
        a = attr_ref[p * SB:(p + 1) * SB, :]
        h = _elu(jnp.dot(a, w1_ref[...], preferred_element_type=jnp.float32)
                 + b1_ref[...])
        h = _elu(jnp.dot(h, w2_ref[...], preferred_element_type=jnp.float32)
                 + b2_ref[...])
        h = _elu(jnp.dot(h, w3p_ref[...], preferred_element_type=jnp.float32)
                 + b3p_ref[...])
        xs = xsrc_ref[p * SB:(p + 1) * SB, :]
        for o in range(OUT_DIM):
            msg_ref[p * SB:(p + 1) * SB, o:o + 1] = jnp.sum(
                xs * h[:, o * IN_DIM:(o + 1) * IN_DIM], axis=1,
                keepdims=True)


def _make_edge(ne, base0):
    blk0 = base0 // EB

    def call(edge_attr, xsrc, w1, b1r, w2, b2r, w3p, b3pr):
        return pl.pallas_call(
            _edge_body,
            grid=(ne // EB,),
            in_specs=[
                pl.BlockSpec((EB, 4), lambda i: (i + blk0, 0)),
                pl.BlockSpec((EB, IN_DIM), lambda i: (i, 0)),
                pl.BlockSpec((4, HID), lambda i: (0, 0)),
                pl.BlockSpec((1, HID), lambda i: (0, 0)),
                pl.BlockSpec((HID, 1024), lambda i: (0, 0)),
                pl.BlockSpec((1, 1024), lambda i: (0, 0)),
                pl.BlockSpec((1024, IN_DIM * OUT_DIM), lambda i: (0, 0)),
                pl.BlockSpec((1, IN_DIM * OUT_DIM), lambda i: (0, 0)),
            ],
            out_specs=pl.BlockSpec((EB, OUT_DIM), lambda i: (i, 0)),
            out_shape=jax.ShapeDtypeStruct((ne, OUT_DIM), jnp.float32),
        )(edge_attr, xsrc, w1, b1r, w2, b2r, w3p, b3pr)

    return call


_edge_a = _make_edge(EA, 0)
_edge_b = _make_edge(EBB, EA)


def _root_body(x_ref, agga_ref, aggb_ref, wroot_ref, broot_ref, xc_ref,
               e_ref):
    xc = jnp.dot(x_ref[...], wroot_ref[...],
                 preferred_element_type=jnp.float32)
    xc = xc + agga_ref[0] + agga_ref[1] + aggb_ref[0] + aggb_ref[1] \
        + broot_ref[...]
    xc_ref[...] = xc
    e_ref[...] = _elu(xc)


def _root(x, agga, aggb, wroot, brootr):
    grid = (N // NB,)
    return pl.pallas_call(
        _root_body,
        grid=grid,
        in_specs=[
            pl.BlockSpec((NB, IN_DIM), lambda i: (i, 0)),
            pl.BlockSpec((NC, NB, OUT_DIM), lambda i: (0, i, 0)),
            pl.BlockSpec((NC, NB, OUT_DIM), lambda i: (0, i, 0)),
            pl.BlockSpec((IN_DIM, OUT_DIM), lambda i: (0, 0)),
            pl.BlockSpec((1, OUT_DIM), lambda i: (0, 0)),
        ],
        out_specs=[
            pl.BlockSpec((NB, OUT_DIM), lambda i: (i, 0)),
            pl.BlockSpec((NB, OUT_DIM), lambda i: (i, 0)),
        ],
        out_shape=[
            jax.ShapeDtypeStruct((N, OUT_DIM), jnp.float32),
            jax.ShapeDtypeStruct((N, OUT_DIM), jnp.float32),
        ],
    )(x, agga, aggb, wroot, brootr)


def _gin_body(xin_ref, nagg_ref, a1_ref, c1_ref, a2_ref, c2_ref, a3_ref,
              c3_ref, out_ref, e_ref):
    h = xin_ref[...] + nagg_ref[0] + nagg_ref[1]
    h = _elu(jnp.dot(h, a1_ref[...], preferred_element_type=jnp.float32)
             + c1_ref[...])
    h = _elu(jnp.dot(h, a2_ref[...], preferred_element_type=jnp.float32)
             + c2_ref[...])
    h = jnp.dot(h, a3_ref[...], preferred_element_type=jnp.float32) \
        + c3_ref[...]
    out_ref[...] = h
    e_ref[...] = _elu(h)


def _gin(xin, nagg, a1, c1r, a2, c2r, a3, c3r):
    grid = (N // NB,)
    return pl.pallas_call(
        _gin_body,
        grid=grid,
        in_specs=[
            pl.BlockSpec((NB, OUT_DIM), lambda i: (i, 0)),
            pl.BlockSpec((NC, NB, OUT_DIM), lambda i: (0, i, 0)),
            pl.BlockSpec((OUT_DIM, HID), lambda i: (0, 0)),
            pl.BlockSpec((1, HID), lambda i: (0, 0)),
            pl.BlockSpec((HID, HID), lambda i: (0, 0)),
            pl.BlockSpec((1, HID), lambda i: (0, 0)),
            pl.BlockSpec((HID, OUT_DIM), lambda i: (0, 0)),
            pl.BlockSpec((1, OUT_DIM), lambda i: (0, 0)),
        ],
        out_specs=[
            pl.BlockSpec((NB, OUT_DIM), lambda i: (i, 0)),
            pl.BlockSpec((NB, OUT_DIM), lambda i: (i, 0)),
        ],
        out_shape=[
            jax.ShapeDtypeStruct((N, OUT_DIM), jnp.float32),
            jax.ShapeDtypeStruct((N, OUT_DIM), jnp.float32),
        ],
    )(xin, nagg, a1, c1r, a2, c2r, a3, c3r)


def kernel(x, edge_index, edge_attr, w1, b1, w2, b2, w3, b3, wroot, broot,
           g1_w1, g1_b1, g1_w2, g1_b2, g1_w3, g1_b3, g2_w1, g2_b1, g2_w2,
           g2_b2, g2_w3, g2_b3):
    src = edge_index[0]
    dst = edge_index[1]

    w3p = w3.reshape(1024, IN_DIM, OUT_DIM).transpose(0, 2, 1) \
        .reshape(1024, IN_DIM * OUT_DIM)
    b3p = b3.reshape(IN_DIM, OUT_DIM).T.reshape(1, IN_DIM * OUT_DIM)
    b1r = b1.reshape(1, -1)
    b2r = b2.reshape(1, -1)

    xsa = _gather_a(x, src)
    msga = _edge_a(edge_attr, xsa, w1, b1r, w2, b2r, w3p, b3p)
    agga = _scatter_a(msga, dst)
    xsb = _gather_b(x, src)
    msgb = _edge_b(edge_attr, xsb, w1, b1r, w2, b2r, w3p, b3p)
    aggb = _scatter_b(msgb, dst)
    xc0, e0 = _root(x, agga, aggb, wroot, broot.reshape(1, -1))

    nagg1 = _sc_gs(e0, src, dst)
    xc1, e1 = _gin(e0, nagg1, g1_w1, g1_b1.reshape(1, -1), g1_w2,
                   g1_b2.reshape(1, -1), g1_w3, g1_b3.reshape(1, -1))

    nagg2 = _sc_gs(e1, src, dst)
    xc2, _ = _gin(e1, nagg2, g2_w1, g2_b1.reshape(1, -1), g2_w2,
                  g2_b2.reshape(1, -1), g2_w3, g2_b3.reshape(1, -1))

    return jnp.stack([xc0, xc1, xc2], axis=2)


# speedup vs baseline: 1.0438x; 1.0024x over previous
import functools

import jax
import jax.numpy as jnp
from jax import lax
from jax.experimental import pallas as pl
from jax.experimental.pallas import tpu as pltpu
from jax.experimental.pallas import tpu_sc as plsc

N = 10000
E = 160000
IN_DIM = 128
OUT_DIM = 16
HID = 256

NC = 2
NS = 16
NW = NC * NS

EA = 32000
EBB = E - EA

E_PER_CORE = E // NC
E_PER_TILE = E_PER_CORE // NS
N_PAD = 10240
N_PER_TILE = N_PAD // NS

GCHUNK = 200
SCHUNK = 1000

EB = 2000
NB = 1000

_MESH = plsc.VectorSubcoreMesh(core_axis_name="c", subcore_axis_name="s")


def _elu(v):
    return jnp.where(v > 0, v, jnp.exp(v) - 1.0)


def _make_gather(ne, base0, gchunk=GCHUNK):
    e_w = ne // NW
    n_ch = e_w // gchunk

    def body(x_hbm, src_hbm, out_hbm, idx_v, rows_v, sem_g, sem_o):
        c = lax.axis_index("c")
        s = lax.axis_index("s")
        wid = s * NC + c
        lbase = wid * e_w
        pltpu.sync_copy(src_hbm.at[pl.ds(base0 + lbase, e_w)], idx_v)
        gd = [None] * n_ch
        od = [None] * n_ch
        for k in range(n_ch):
            if k >= 2:
                od[k - 2].wait()
            gd[k] = pltpu.async_copy(
                x_hbm.at[idx_v.at[pl.ds(k * gchunk, gchunk)]],
                rows_v.at[k % 2], sem_g)
            if k >= 1:
                gd[k - 1].wait()
                od[k - 1] = pltpu.async_copy(
                    rows_v.at[(k - 1) % 2],
                    out_hbm.at[pl.ds(lbase + (k - 1) * gchunk, gchunk)],
                    sem_o)
        gd[n_ch - 1].wait()
        od[n_ch - 1] = pltpu.async_copy(
            rows_v.at[(n_ch - 1) % 2],
            out_hbm.at[pl.ds(lbase + (n_ch - 1) * gchunk, gchunk)], sem_o)
        if n_ch >= 2:
            od[n_ch - 2].wait()
        od[n_ch - 1].wait()

    return functools.partial(
        pl.kernel,
        out_type=jax.ShapeDtypeStruct((ne, IN_DIM), jnp.float32),
        mesh=_MESH,
        scratch_types=[
            pltpu.VMEM((e_w,), jnp.int32),
            pltpu.VMEM((2, gchunk, IN_DIM), jnp.float32),
            pltpu.SemaphoreType.DMA,
            pltpu.SemaphoreType.DMA,
        ],
    )(body)


_gather_a = _make_gather(EA, 0)
_gather_b = _make_gather(EBB, EA, gchunk=400)


def _make_scatter(ne, base0):
    e_core = ne // NC
    e_tile = e_core // NS
    n_ch = e_tile // SCHUNK

    def body(msg_hbm, dst_hbm, out_hbm, acc_sh, idx_v, val_v, zrow_v, sem_g,
             sem_s):
        c = lax.axis_index("c")
        s = lax.axis_index("s")

        def zfill(i, carry):
            zrow_v[i, :] = jnp.zeros((OUT_DIM,), jnp.float32)
            return carry

        lax.fori_loop(0, N_PER_TILE, zfill, 0)
        pltpu.sync_copy(zrow_v, acc_sh.at[pl.ds(s * N_PER_TILE, N_PER_TILE)])

        lbase = c * e_core + s * e_tile
        for k in range(n_ch):
            pltpu.sync_copy(
                dst_hbm.at[pl.ds(base0 + lbase + k * SCHUNK, SCHUNK)],
                idx_v.at[k])
        plsc.subcore_barrier()

        gd = [None] * n_ch
        sd = [None] * n_ch
        for k in range(n_ch):
            if k >= 2:
                sd[k - 2].wait()
            gd[k] = pltpu.async_copy(
                msg_hbm.at[pl.ds(lbase + k * SCHUNK, SCHUNK)],
                val_v.at[k % 2], sem_g)
            if k >= 1:
                gd[k - 1].wait()
                sd[k - 1] = pltpu.async_copy(
                    val_v.at[(k - 1) % 2], acc_sh.at[idx_v.at[k - 1]],
                    sem_s, add=True)
        gd[n_ch - 1].wait()
        sd[n_ch - 1] = pltpu.async_copy(
            val_v.at[(n_ch - 1) % 2], acc_sh.at[idx_v.at[n_ch - 1]], sem_s,
            add=True)
        if n_ch >= 2:
            sd[n_ch - 2].wait()
        sd[n_ch - 1].wait()

        plsc.subcore_barrier()
        pltpu.sync_copy(acc_sh.at[pl.ds(s * N_PER_TILE, N_PER_TILE)],
                        out_hbm.at[c, pl.ds(s * N_PER_TILE, N_PER_TILE)])

    return functools.partial(
        pl.kernel,
        out_type=jax.ShapeDtypeStruct((NC, N_PAD, OUT_DIM), jnp.float32),
        mesh=_MESH,
        compiler_params=pltpu.CompilerParams(use_tc_tiling_on_sc=False),
        scratch_types=[
            pltpu.VMEM_SHARED((N_PAD, OUT_DIM), jnp.float32),
            pltpu.VMEM((n_ch, SCHUNK), jnp.int32),
            pltpu.VMEM((2, SCHUNK, OUT_DIM), jnp.float32),
            pltpu.VMEM((N_PER_TILE, OUT_DIM), jnp.float32),
            pltpu.SemaphoreType.DMA,
            pltpu.SemaphoreType.DMA,
        ],
    )(body)


_scatter_a = _make_scatter(EA, 0)
_scatter_b = _make_scatter(EBB, EA)


N_GSCH = E_PER_TILE // SCHUNK


def _sc_gs_body(xin_hbm, src_hbm, dst_hbm, out_hbm, acc_sh, sidx_v, didx_v,
                val_v, zrow_v, sem_g, sem_s):
    c = lax.axis_index("c")
    s = lax.axis_index("s")

    def zfill(i, carry):
        zrow_v[i, :] = jnp.zeros((OUT_DIM,), jnp.float32)
        return carry

    lax.fori_loop(0, N_PER_TILE, zfill, 0)
    pltpu.sync_copy(zrow_v, acc_sh.at[pl.ds(s * N_PER_TILE, N_PER_TILE)])

    base = c * E_PER_CORE + s * E_PER_TILE
    pltpu.sync_copy(src_hbm.at[pl.ds(base, E_PER_TILE)], sidx_v)
    for k in range(N_GSCH):
        pltpu.sync_copy(dst_hbm.at[pl.ds(base + k * SCHUNK, SCHUNK)],
                        didx_v.at[k])
    plsc.subcore_barrier()

    gd = [None] * N_GSCH
    sd = [None] * N_GSCH
    for k in range(N_GSCH):
        if k >= 2:
            sd[k - 2].wait()
        gd[k] = pltpu.async_copy(
            xin_hbm.at[sidx_v.at[pl.ds(k * SCHUNK, SCHUNK)]],
            val_v.at[k % 2], sem_g)
        if k >= 1:
            gd[k - 1].wait()
            sd[k - 1] = pltpu.async_copy(
                val_v.at[(k - 1) % 2], acc_sh.at[didx_v.at[k - 1]], sem_s,
                add=True)
    gd[N_GSCH - 1].wait()
    sd[N_GSCH - 1] = pltpu.async_copy(
        val_v.at[(N_GSCH - 1) % 2], acc_sh.at[didx_v.at[N_GSCH - 1]], sem_s,
        add=True)
    if N_GSCH >= 2:
        sd[N_GSCH - 2].wait()
    sd[N_GSCH - 1].wait()

    plsc.subcore_barrier()
    pltpu.sync_copy(acc_sh.at[pl.ds(s * N_PER_TILE, N_PER_TILE)],
                    out_hbm.at[c, pl.ds(s * N_PER_TILE, N_PER_TILE)])


_sc_gs = functools.partial(
    pl.kernel,
    out_type=jax.ShapeDtypeStruct((NC, N_PAD, OUT_DIM), jnp.float32),
    mesh=_MESH,
    compiler_params=pltpu.CompilerParams(use_tc_tiling_on_sc=False),
    scratch_types=[
        pltpu.VMEM_SHARED((N_PAD, OUT_DIM), jnp.float32),
        pltpu.VMEM((E_PER_TILE,), jnp.int32),
        pltpu.VMEM((N_GSCH, SCHUNK), jnp.int32),
        pltpu.VMEM((2, SCHUNK, OUT_DIM), jnp.float32),
        pltpu.VMEM((N_PER_TILE, OUT_DIM), jnp.float32),
        pltpu.SemaphoreType.DMA,
        pltpu.SemaphoreType.DMA,
    ],
)(_sc_gs_body)


SB = EB // 2


def _edge_body(attr_ref, xsrc_ref, w1_ref, b1_ref, w2_ref, b2_ref, w3p_ref,
               b3p_ref, msg_ref):
    for p in range(EB // SB):
        a = attr_ref[p * SB:(p + 1) * SB, :]
        h = _elu(jnp.dot(a, w1_ref[...], preferred_element_type=jnp.float32)
                 + b1_ref[...])
        h = _elu(jnp.dot(h, w2_ref[...], preferred_element_type=jnp.float32)
                 + b2_ref[...])
        h = _elu(jnp.dot(h, w3p_ref[...], preferred_element_type=jnp.float32)
                 + b3p_ref[...])
        xs = xsrc_ref[p * SB:(p + 1) * SB, :]
        for o in range(OUT_DIM):
            msg_ref[p * SB:(p + 1) * SB, o:o + 1] = jnp.sum(
                xs * h[:, o * IN_DIM:(o + 1) * IN_DIM], axis=1,
                keepdims=True)


def _make_edge(ne, base0):
    blk0 = base0 // EB

    def call(edge_attr, xsrc, w1, b1r, w2, b2r, w3p, b3pr):
        return pl.pallas_call(
            _edge_body,
            grid=(ne // EB,),
            in_specs=[
                pl.BlockSpec((EB, 4), lambda i: (i + blk0, 0)),
                pl.BlockSpec((EB, IN_DIM), lambda i: (i, 0)),
                pl.BlockSpec((4, HID), lambda i: (0, 0)),
                pl.BlockSpec((1, HID), lambda i: (0, 0)),
                pl.BlockSpec((HID, 1024), lambda i: (0, 0)),
                pl.BlockSpec((1, 1024), lambda i: (0, 0)),
                pl.BlockSpec((1024, IN_DIM * OUT_DIM), lambda i: (0, 0)),
                pl.BlockSpec((1, IN_DIM * OUT_DIM), lambda i: (0, 0)),
            ],
            out_specs=pl.BlockSpec((EB, OUT_DIM), lambda i: (i, 0)),
            out_shape=jax.ShapeDtypeStruct((ne, OUT_DIM), jnp.float32),
        )(edge_attr, xsrc, w1, b1r, w2, b2r, w3p, b3pr)

    return call


_edge_a = _make_edge(EA, 0)
_edge_b = _make_edge(EBB, EA)


def _root_body(x_ref, agga_ref, aggb_ref, wroot_ref, broot_ref, xc_ref,
               e_ref):
    xc = jnp.dot(x_ref[...], wroot_ref[...],
                 preferred_element_type=jnp.float32)
    xc = xc + agga_ref[0] + agga_ref[1] + aggb_ref[0] + aggb_ref[1] \
        + broot_ref[...]
    xc_ref[...] = xc
    e_ref[...] = _elu(xc)


def _root(x, agga, aggb, wroot, brootr):
    grid = (N // NB,)
    return pl.pallas_call(
        _root_body,
        grid=grid,
        in_specs=[
            pl.BlockSpec((NB, IN_DIM), lambda i: (i, 0)),
            pl.BlockSpec((NC, NB, OUT_DIM), lambda i: (0, i, 0)),
            pl.BlockSpec((NC, NB, OUT_DIM), lambda i: (0, i, 0)),
            pl.BlockSpec((IN_DIM, OUT_DIM), lambda i: (0, 0)),
            pl.BlockSpec((1, OUT_DIM), lambda i: (0, 0)),
        ],
        out_specs=[
            pl.BlockSpec((NB, OUT_DIM), lambda i: (i, 0)),
            pl.BlockSpec((NB, OUT_DIM), lambda i: (i, 0)),
        ],
        out_shape=[
            jax.ShapeDtypeStruct((N, OUT_DIM), jnp.float32),
            jax.ShapeDtypeStruct((N, OUT_DIM), jnp.float32),
        ],
    )(x, agga, aggb, wroot, brootr)


def _gin_body(xin_ref, nagg_ref, a1_ref, c1_ref, a2_ref, c2_ref, a3_ref,
              c3_ref, out_ref, e_ref):
    h = xin_ref[...] + nagg_ref[0] + nagg_ref[1]
    h = _elu(jnp.dot(h, a1_ref[...], preferred_element_type=jnp.float32)
             + c1_ref[...])
    h = _elu(jnp.dot(h, a2_ref[...], preferred_element_type=jnp.float32)
             + c2_ref[...])
    h = jnp.dot(h, a3_ref[...], preferred_element_type=jnp.float32) \
        + c3_ref[...]
    out_ref[...] = h
    e_ref[...] = _elu(h)


def _gin(xin, nagg, a1, c1r, a2, c2r, a3, c3r):
    grid = (N // NB,)
    return pl.pallas_call(
        _gin_body,
        grid=grid,
        in_specs=[
            pl.BlockSpec((NB, OUT_DIM), lambda i: (i, 0)),
            pl.BlockSpec((NC, NB, OUT_DIM), lambda i: (0, i, 0)),
            pl.BlockSpec((OUT_DIM, HID), lambda i: (0, 0)),
            pl.BlockSpec((1, HID), lambda i: (0, 0)),
            pl.BlockSpec((HID, HID), lambda i: (0, 0)),
            pl.BlockSpec((1, HID), lambda i: (0, 0)),
            pl.BlockSpec((HID, OUT_DIM), lambda i: (0, 0)),
            pl.BlockSpec((1, OUT_DIM), lambda i: (0, 0)),
        ],
        out_specs=[
            pl.BlockSpec((NB, OUT_DIM), lambda i: (i, 0)),
            pl.BlockSpec((NB, OUT_DIM), lambda i: (i, 0)),
        ],
        out_shape=[
            jax.ShapeDtypeStruct((N, OUT_DIM), jnp.float32),
            jax.ShapeDtypeStruct((N, OUT_DIM), jnp.float32),
        ],
    )(xin, nagg, a1, c1r, a2, c2r, a3, c3r)


def kernel(x, edge_index, edge_attr, w1, b1, w2, b2, w3, b3, wroot, broot,
           g1_w1, g1_b1, g1_w2, g1_b2, g1_w3, g1_b3, g2_w1, g2_b1, g2_w2,
           g2_b2, g2_w3, g2_b3):
    src = edge_index[0]
    dst = edge_index[1]

    w3p = w3.reshape(1024, IN_DIM, OUT_DIM).transpose(0, 2, 1) \
        .reshape(1024, IN_DIM * OUT_DIM)
    b3p = b3.reshape(IN_DIM, OUT_DIM).T.reshape(1, IN_DIM * OUT_DIM)
    b1r = b1.reshape(1, -1)
    b2r = b2.reshape(1, -1)

    xsa = _gather_a(x, src)
    msga = _edge_a(edge_attr, xsa, w1, b1r, w2, b2r, w3p, b3p)
    agga = _scatter_a(msga, dst)
    xsb = _gather_b(x, src)
    msgb = _edge_b(edge_attr, xsb, w1, b1r, w2, b2r, w3p, b3p)
    aggb = _scatter_b(msgb, dst)
    xc0, e0 = _root(x, agga, aggb, wroot, broot.reshape(1, -1))

    nagg1 = _sc_gs(e0, src, dst)
    xc1, e1 = _gin(e0, nagg1, g1_w1, g1_b1.reshape(1, -1), g1_w2,
                   g1_b2.reshape(1, -1), g1_w3, g1_b3.reshape(1, -1))

    nagg2 = _sc_gs(e1, src, dst)
    xc2, _ = _gin(e1, nagg2, g2_w1, g2_b1.reshape(1, -1), g2_w2,
                  g2_b2.reshape(1, -1), g2_w3, g2_b3.reshape(1, -1))

    return jnp.stack([xc0, xc1, xc2], axis=2)
